# baseline (speedup 1.0000x reference)
import jax
import jax.numpy as jnp
from jax.experimental import pallas as pl
from jax.experimental.pallas import tpu as pltpu

_E = 8
_N_GROUP = 4
_D = 2048
_DI = 1408
_DI_S = 2816
_ROUTED_SCALING = 2.5
_T = 512
_BLK = 128
_MAX_RB = 16
_DIT_S = 256
_NDI_S = _DI_S // _DIT_S
_DIT = 128
_NDI = _DI // _DIT

_F32 = jnp.float32
_BF16 = jnp.bfloat16
_I32 = jnp.int32
_EXACT = jax.lax.Precision.HIGHEST


def _routing_body(h_ref, gw_ref, bias_ref, m_ref, wc_ref, rex_ref, nrb_ref):
    logits = jax.lax.dot_general(h_ref[...], gw_ref[...],
                                 (((1,), (1,)), ((), ())),
                                 preferred_element_type=_F32)
    s = jax.nn.sigmoid(logits)
    sb = s + bias_ref[...]

    gsz = _E // _N_GROUP
    e_i = jax.lax.broadcasted_iota(_I32, (_E, _N_GROUP), 0)
    g_i = jax.lax.broadcasted_iota(_I32, (_E, _N_GROUP), 1)
    grp_mat = (e_i // gsz == g_i).astype(_F32)
    gs = jnp.dot(sb, grp_mat, preferred_element_type=_F32,
                 precision=_EXACT)

    g_lane = jax.lax.broadcasted_iota(_I32, (_T, _N_GROUP), 1)
    g1 = jnp.argmax(gs, axis=1)[:, None]
    m_g1 = g_lane == g1
    g2 = jnp.argmax(jnp.where(m_g1, -jnp.inf, gs), axis=1)[:, None]
    gmask = (m_g1 | (g_lane == g2)).astype(_F32)
    emask = jnp.dot(gmask, grp_mat.T, preferred_element_type=_F32,
                    precision=_EXACT)

    e_lane = jax.lax.broadcasted_iota(_I32, (_T, _E), 1)
    masked = jnp.where(emask > 0, sb, -jnp.inf)
    e1 = jnp.argmax(masked, axis=1)[:, None]
    m_e1 = e_lane == e1
    e2 = jnp.argmax(jnp.where(m_e1, -jnp.inf, masked), axis=1)[:, None]
    sel = m_e1 | (e_lane == e2)
    sel_f = sel.astype(_F32)

    w_raw = s * sel_f
    w_sum = jnp.sum(w_raw, axis=1, keepdims=True)
    w_n = (w_raw / w_sum).astype(_BF16).astype(_F32)
    wc_ref[...] = w_n * _ROUTED_SCALING

    t_i = jax.lax.broadcasted_iota(_I32, (_T, _T), 0)
    t_j = jax.lax.broadcasted_iota(_I32, (_T, _T), 1)
    ltri = (t_j < t_i).astype(_F32)
    rank = jnp.dot(ltri, sel_f, preferred_element_type=_F32,
                   precision=_EXACT).astype(_I32)
    counts = jnp.sum(sel_f, axis=0, keepdims=True).astype(_I32)
    nblk = (counts + _BLK - 1) // _BLK
    ee_i = jax.lax.broadcasted_iota(_I32, (_E, _E), 0)
    ee_j = jax.lax.broadcasted_iota(_I32, (_E, _E), 1)
    nblk_f = nblk.astype(_F32)
    blk_start = jnp.dot(nblk_f, (ee_i < ee_j).astype(_F32),
                        preferred_element_type=_F32,
                        precision=_EXACT).astype(_I32)
    cum_end = jnp.dot(nblk_f, (ee_i <= ee_j).astype(_F32),
                      preferred_element_type=_F32,
                      precision=_EXACT).astype(_I32)
    pad_start = _BLK * blk_start
    m_ref[...] = jnp.where(sel, pad_start + rank, -1)

    total = jnp.max(cum_end)
    nrb_ref[...] = jnp.reshape(total, (1, 1))
    b_i = jax.lax.broadcasted_iota(_I32, (_MAX_RB, _E), 0)
    ex = jnp.sum((b_i >= cum_end).astype(_I32), axis=1, keepdims=True)
    last_ex = jnp.sum(((total - 1) >= cum_end).astype(_I32))
    b_col = jax.lax.broadcasted_iota(_I32, (_MAX_RB, 1), 0)
    rex_ref[...] = jnp.where(b_col < total, ex, last_ex)


def _routing(h16, gw16, bias2):
    return pl.pallas_call(
        _routing_body,
        out_shape=(
            jax.ShapeDtypeStruct((_T, _E), _I32),
            jax.ShapeDtypeStruct((_T, _E), _F32),
            jax.ShapeDtypeStruct((_MAX_RB, 1), _I32),
            jax.ShapeDtypeStruct((1, 1), _I32),
        ),
    )(h16, gw16, bias2)


def _shared_body(h_ref, wg_ref, wu_ref, wd_ref, out_ref):
    @pl.when(pl.program_id(0) == 0)
    def _():
        out_ref[...] = jnp.zeros_like(out_ref)

    h = h_ref[...]
    g = jnp.dot(h, wg_ref[...], preferred_element_type=_F32)
    u = jnp.dot(h, wu_ref[...], preferred_element_type=_F32)
    a = g * jax.nn.sigmoid(g) * u
    out_ref[...] += jnp.dot(a.astype(_BF16), wd_ref[...],
                            preferred_element_type=_F32)


def _shared(h16, wsgu16, wsd16):
    return pl.pallas_call(
        _shared_body,
        grid=(_NDI_S,),
        in_specs=[
            pl.BlockSpec((_T, _D), lambda di: (0, 0)),
            pl.BlockSpec((_D, _DIT_S), lambda di: (0, di)),
            pl.BlockSpec((_D, _DIT_S), lambda di: (0, di + _NDI_S)),
            pl.BlockSpec((_DIT_S, _D), lambda di: (di, 0)),
        ],
        out_specs=pl.BlockSpec((_T, _D), lambda di: (0, 0)),
        out_shape=jax.ShapeDtypeStruct((_T, _D), _F32),
        compiler_params=pltpu.CompilerParams(
            dimension_semantics=("arbitrary",)),
    )(h16, wsgu16, wsgu16, wsd16)


def _routed_body(nrb_ref, rex_ref, m_ref, wc_ref, h_ref, base_ref,
                 wg_ref, wu_ref, wd_ref, out_ref, hb_ref, y_ref):
    b = pl.program_id(0)
    di = pl.program_id(1)

    @pl.when((b == 0) & (di == 0))
    def _():
        out_ref[...] = base_ref[...]

    @pl.when(b < nrb_ref[0])
    def _():
        ex = rex_ref[b]
        lane = jax.lax.broadcasted_iota(_I32, (_T, _E), 1)
        exmask = lane == ex
        m_col = jnp.sum(jnp.where(exmask, m_ref[...], 0), axis=1,
                        keepdims=True)
        slots = _BLK * b + jax.lax.broadcasted_iota(_I32, (1, _BLK), 1)
        oht = (m_col == slots).astype(_F32)

        @pl.when(di == 0)
        def _():
            hb_ref[...] = jax.lax.dot_general(
                oht.astype(_BF16), h_ref[...], (((0,), (0,)), ((), ())),
                preferred_element_type=_F32)

        hb = hb_ref[...].astype(_BF16)
        g = jnp.dot(hb, wg_ref[0], preferred_element_type=_F32)
        u = jnp.dot(hb, wu_ref[0], preferred_element_type=_F32)
        a = g * jax.nn.sigmoid(g) * u
        y_di = jnp.dot(a.astype(_BF16), wd_ref[0],
                       preferred_element_type=_F32)

        @pl.when(di == 0)
        def _():
            y_ref[...] = jnp.zeros_like(y_ref)

        y_ref[...] += y_di

        @pl.when(di == _NDI - 1)
        def _():
            w_col = jnp.sum(jnp.where(exmask, wc_ref[...], 0.0), axis=1,
                            keepdims=True)
            w_row = jax.lax.dot_general(
                oht, w_col, (((0,), (0,)), ((), ())),
                preferred_element_type=_F32, precision=_EXACT)
            yw = y_ref[...].astype(_BF16).astype(_F32) * w_row
            for c in range(_T // _BLK):
                out_ref[c * _BLK:(c + 1) * _BLK, :] += jnp.dot(
                    oht[c * _BLK:(c + 1) * _BLK, :], yw,
                    preferred_element_type=_F32, precision=_EXACT)


def _routed(nrb, rex, m, wc, h16, base, wgu16, wd16):
    spec = pltpu.PrefetchScalarGridSpec(
        num_scalar_prefetch=2,
        grid=(_MAX_RB, _NDI),
        in_specs=[
            pl.BlockSpec((_T, _E), lambda b, di, nrb, rex: (0, 0)),
            pl.BlockSpec((_T, _E), lambda b, di, nrb, rex: (0, 0)),
            pl.BlockSpec((_T, _D), lambda b, di, nrb, rex: (0, 0)),
            pl.BlockSpec((_T, _D), lambda b, di, nrb, rex: (0, 0)),
            pl.BlockSpec((1, _D, _DIT),
                         lambda b, di, nrb, rex: (rex[b], 0, di)),
            pl.BlockSpec((1, _D, _DIT),
                         lambda b, di, nrb, rex: (rex[b], 0, di + _NDI)),
            pl.BlockSpec((1, _DIT, _D),
                         lambda b, di, nrb, rex: (rex[b], di, 0)),
        ],
        out_specs=pl.BlockSpec((_T, _D), lambda b, di, nrb, rex: (0, 0)),
        scratch_shapes=[
            pltpu.VMEM((_BLK, _D), _F32),
            pltpu.VMEM((_BLK, _D), _F32),
        ],
    )
    return pl.pallas_call(
        _routed_body,
        grid_spec=spec,
        out_shape=jax.ShapeDtypeStruct((_T, _D), _F32),
        compiler_params=pltpu.CompilerParams(
            dimension_semantics=("arbitrary", "arbitrary")),
    )(nrb, rex, m, wc, h16, base, wgu16, wgu16, wd16)


def kernel(hidden_states, gate_weight, e_score_correction_bias, w_gate_up,
           w_down, ws_gate_up, ws_down, num_global_tokens,
           max_num_tokens_per_gpu):
    h16 = hidden_states.astype(_BF16)
    gw16 = gate_weight.astype(_BF16)
    wgu16 = w_gate_up.astype(_BF16)
    wd16 = w_down.astype(_BF16)
    wsgu16 = ws_gate_up.astype(_BF16)
    wsd16 = ws_down.astype(_BF16)
    bias2 = e_score_correction_bias.reshape(1, _E)
    m, wc, rex, nrb = _routing(h16, gw16, bias2)
    shared = _shared(h16, wsgu16, wsd16)
    return _routed(nrb.reshape(-1), rex.reshape(-1), m, wc, h16,
                   shared, wgu16, wd16)
